# baseline (speedup 1.0000x reference)
import functools

import jax
import jax.numpy as jnp
from jax import lax
from jax.experimental import pallas as pl
from jax.experimental.pallas import tpu as pltpu
from jax.experimental.pallas import tpu_sc as plsc

N_NODES = 50000
D = 128
N_EDGES = 800000

NPAD = 50176
E_PAD = 802816
NT = 16
NC = 2
ROWS_PER_TILE = NPAD // NT
Q = 196
UW = 128
NU_MAIN = 392
NG_MAIN = 14
NUG = NU_MAIN // NG_MAIN
NBUF = 4
DEPTH = 3
NB_DEG = 196
NG_DEG = 2
NBG_DEG = NB_DEG // NG_DEG

_MESH = plsc.VectorSubcoreMesh(
    core_axis_name="c", subcore_axis_name="s", num_cores=NC, num_subcores=NT)
_f32 = jnp.float32


@functools.partial(
    pl.kernel,
    out_type=jax.ShapeDtypeStruct((NC, NPAD, 16), _f32),
    mesh=_MESH,
    compiler_params=pltpu.CompilerParams(use_tc_tiling_on_sc=False),
    scratch_types=[
        pltpu.VMEM_SHARED((NPAD, 16), _f32),
        pltpu.VMEM((NBG_DEG, 128), jnp.int32),
        pltpu.VMEM((128, 16), _f32),
    ],
)
def _deg_kernel(dst3, ones_h, zeros_h, out, accum, dst_v, ones_v):
    c = lax.axis_index("c")
    s = lax.axis_index("s")
    w = s * NC + c
    tsl = pl.ds(s * ROWS_PER_TILE, ROWS_PER_TILE)
    pltpu.sync_copy(ones_h, ones_v)
    pltpu.sync_copy(zeros_h, accum.at[tsl])
    plsc.subcore_barrier()

    for g in range(NG_DEG):
        pltpu.sync_copy(dst3.at[w, pl.ds(g * NBG_DEG, NBG_DEG)], dst_v)

        def body(b, carry):
            pltpu.sync_copy(ones_v, accum.at[dst_v.at[b]], add=True)
            return carry

        lax.fori_loop(0, NBG_DEG, body, 0)
    plsc.subcore_barrier()
    pltpu.sync_copy(accum.at[tsl], out.at[c, tsl])


@functools.partial(
    pl.kernel,
    out_type=jax.ShapeDtypeStruct((NPAD, D), _f32),
    mesh=_MESH,
    compiler_params=pltpu.CompilerParams(use_tc_tiling_on_sc=False),
    scratch_types=[
        pltpu.VMEM_SHARED((NPAD, 32), _f32),
        pltpu.VMEM((NUG, 2, UW), jnp.int32),
    ] + [pltpu.VMEM((UW, 32), _f32)] * NBUF
      + [pltpu.SemaphoreType.DMA] * (2 * NBUF),
)
def _scatter_kernel(t_h, sd4, zeros_h, o4,
                    accum, idx_v, *bufsem):
    bufs = bufsem[:NBUF]
    gsem = bufsem[NBUF:2 * NBUF]
    ssem = bufsem[2 * NBUF:]
    c = lax.axis_index("c")
    s = lax.axis_index("s")
    tsl = pl.ds(s * ROWS_PER_TILE, ROWS_PER_TILE)

    for kk in range(2):
        chunk = 2 * c + kk
        pltpu.sync_copy(zeros_h, accum.at[tsl])
        plsc.subcore_barrier()

        def fire(u):
            b = u % NBUF
            pltpu.async_copy(t_h.at[idx_v.at[u, 0]], bufs[b], gsem[b])

        def wait_g(u):
            b = u % NBUF
            pltpu.make_async_copy(
                t_h.at[idx_v.at[u, 0]], bufs[b], gsem[b]).wait()

        def scat(u):
            b = u % NBUF
            pltpu.async_copy(bufs[b], accum.at[idx_v.at[u, 1]], ssem[b],
                             add=True)

        def wait_s(u):
            b = u % NBUF
            pltpu.make_async_copy(
                bufs[b], accum.at[idx_v.at[u, 1]], ssem[b]).wait()

        def group(g, carry):
            pltpu.sync_copy(sd4.at[chunk, s, pl.ds(g * NUG, NUG)], idx_v)
            later = g > 0
            for x in range(DEPTH):
                @pl.when(later)
                def _(x=x):
                    wait_s(NUG - NBUF + x)

                fire(x)

            for u in range(NUG):
                nf = u + DEPTH
                if nf < NUG:
                    if u > 0:
                        wait_s(u - 1)
                    else:
                        @pl.when(later)
                        def _():
                            wait_s(NUG - 1)

                    fire(nf)
                wait_g(u)
                scat(u)
            return carry

        lax.fori_loop(0, NG_MAIN, group, 0)
        for u in range(NUG - NBUF, NUG):
            wait_s(u)
        plsc.subcore_barrier()
        pltpu.sync_copy(accum.at[tsl], o4.at[tsl, pl.ds(32 * chunk, 32)])


_BLK = 512
_GRID = NPAD // _BLK


def _dinv_block(degp):
    dsum = degp[0] + degp[1]
    return lax.rsqrt(dsum[:, 0:1] + 1.0)


_NAT = pl.BlockSpec((_BLK, D), lambda i: (i, 0))
_NAT_SHAPE = jax.ShapeDtypeStruct((NPAD, D), _f32)
_DEGP = pl.BlockSpec((NC, _BLK, 16), lambda i: (0, i, 0))


def _prescale_body(emb_ref, degp_ref, g):
    dinv = _dinv_block(degp_ref[...])
    g[...] = emb_ref[...] * dinv


def _tc_prescale(emb_pad, degp):
    return pl.pallas_call(
        _prescale_body,
        grid=(_GRID,),
        in_specs=[_NAT, _DEGP],
        out_specs=_NAT,
        out_shape=_NAT_SHAPE,
    )(emb_pad, degp)


def _mid_body(s, g, degp_ref, w1, b1, w2, f):
    dinv = _dinv_block(degp_ref[...])
    p1 = dinv * (s[...] + g[...])
    z = jnp.maximum(
        jnp.dot(p1, w1[...], preferred_element_type=_f32) + b1[...], 0.0)
    h2 = jnp.dot(z, w2[...], preferred_element_type=_f32)
    f[...] = h2 * dinv


def _tc_mid(s, g, degp, W1, b1, W2):
    return pl.pallas_call(
        _mid_body,
        grid=(_GRID,),
        in_specs=[_NAT, _NAT, _DEGP,
                  pl.BlockSpec((D, 2 * D), lambda i: (0, 0)),
                  pl.BlockSpec((1, 2 * D), lambda i: (0, 0)),
                  pl.BlockSpec((2 * D, D), lambda i: (0, 0))],
        out_specs=_NAT,
        out_shape=_NAT_SHAPE,
    )(s, g, degp, W1, b1.reshape(1, 2 * D), W2)


def _final_body(s, f, degp_ref, b2, out):
    dinv = _dinv_block(degp_ref[...])
    out[...] = dinv * (s[...] + f[...]) + b2[...]


def _tc_final(s, f, degp, b2):
    return pl.pallas_call(
        _final_body,
        grid=(_GRID,),
        in_specs=[_NAT, _NAT, _DEGP,
                  pl.BlockSpec((1, D), lambda i: (0, 0))],
        out_specs=pl.BlockSpec((_BLK, D), lambda i: (i, 0)),
        out_shape=jax.ShapeDtypeStruct((N_NODES, D), _f32),
    )(s, f, degp, b2.reshape(1, D))


def kernel(edge_index, emb, W1, b1, W2, b2):
    src = edge_index[0].astype(jnp.int32)
    dst = edge_index[1].astype(jnp.int32)
    npad_extra = E_PAD - N_EDGES
    pad_ids = jnp.arange(npad_extra, dtype=jnp.int32)
    src_p = jnp.concatenate([src, pad_ids % N_NODES])
    dst_p = jnp.concatenate([dst, N_NODES + pad_ids % (NPAD - N_NODES)])
    src24 = (4 * src_p[None, :] + jnp.arange(4, dtype=jnp.int32)[:, None]
             ).reshape(4, NT, NU_MAIN, 1, UW)
    dst2 = jnp.broadcast_to(
        dst_p.reshape(1, NT, NU_MAIN, 1, UW), (4, NT, NU_MAIN, 1, UW))
    sd4 = jnp.concatenate([src24, dst2], axis=3)
    dst3 = dst_p.reshape(NT * NC, NB_DEG, 128)

    emb_pad = jnp.pad(emb, ((0, NPAD - N_NODES), (0, 0)))
    ones16 = jnp.ones((128, 16), _f32)
    z16 = jnp.zeros((ROWS_PER_TILE, 16), _f32)
    z32 = jnp.zeros((ROWS_PER_TILE, 32), _f32)

    degp = _deg_kernel(dst3, ones16, z16)
    g = _tc_prescale(emb_pad, degp)
    s = _scatter_kernel(g.reshape(4 * NPAD, 32), sd4, z32)
    f = _tc_mid(s, g, degp, W1, b1, W2)
    t = _scatter_kernel(f.reshape(4 * NPAD, 32), sd4, z32)
    return _tc_final(t, f, degp, b2)


# speedup vs baseline: 1.0013x; 1.0013x over previous
import functools

import jax
import jax.numpy as jnp
from jax import lax
from jax.experimental import pallas as pl
from jax.experimental.pallas import tpu as pltpu
from jax.experimental.pallas import tpu_sc as plsc

N_NODES = 50000
D = 128
N_EDGES = 800000

NPAD = 50176
E_PAD = 802816
NT = 16
NC = 2
ROWS_PER_TILE = NPAD // NT
Q = 196
UW = 128
NU_MAIN = 392
NG_MAIN = 14
NUG = NU_MAIN // NG_MAIN
NBUF = 4
DEPTH = 3
NB_DEG = 196
NG_DEG = 2
NBG_DEG = NB_DEG // NG_DEG

_MESH = plsc.VectorSubcoreMesh(
    core_axis_name="c", subcore_axis_name="s", num_cores=NC, num_subcores=NT)
_f32 = jnp.float32


@functools.partial(
    pl.kernel,
    out_type=jax.ShapeDtypeStruct((NC, NPAD, 16), _f32),
    mesh=_MESH,
    compiler_params=pltpu.CompilerParams(use_tc_tiling_on_sc=False),
    scratch_types=[
        pltpu.VMEM_SHARED((NPAD, 16), _f32),
        pltpu.VMEM((NBG_DEG, 128), jnp.int32),
        pltpu.VMEM((128, 16), _f32),
    ],
)
def _deg_kernel(dst3, ones_h, zeros_h, out, accum, dst_v, ones_v):
    c = lax.axis_index("c")
    s = lax.axis_index("s")
    w = s * NC + c
    tsl = pl.ds(s * ROWS_PER_TILE, ROWS_PER_TILE)
    pltpu.sync_copy(ones_h, ones_v)
    pltpu.sync_copy(zeros_h, accum.at[tsl])
    plsc.subcore_barrier()

    for g in range(NG_DEG):
        pltpu.sync_copy(dst3.at[w, pl.ds(g * NBG_DEG, NBG_DEG)], dst_v)

        def body(b, carry):
            pltpu.sync_copy(ones_v, accum.at[dst_v.at[b]], add=True)
            return carry

        lax.fori_loop(0, NBG_DEG, body, 0)
    plsc.subcore_barrier()
    pltpu.sync_copy(accum.at[tsl], out.at[c, tsl])


@functools.partial(
    pl.kernel,
    out_type=jax.ShapeDtypeStruct((NPAD, D), _f32),
    mesh=_MESH,
    compiler_params=pltpu.CompilerParams(use_tc_tiling_on_sc=False),
    scratch_types=[
        pltpu.VMEM_SHARED((NPAD, 32), _f32),
        pltpu.VMEM((NUG, 2, UW), jnp.int32),
    ] + [pltpu.VMEM((UW, 32), _f32)] * NBUF
      + [pltpu.SemaphoreType.DMA] * (2 * NBUF),
)
def _scatter_kernel(t_h, sd4, zeros_h, o4,
                    accum, idx_v, *bufsem):
    bufs = bufsem[:NBUF]
    gsem = bufsem[NBUF:2 * NBUF]
    ssem = bufsem[2 * NBUF:]
    c = lax.axis_index("c")
    s = lax.axis_index("s")
    tsl = pl.ds(s * ROWS_PER_TILE, ROWS_PER_TILE)

    for kk in range(2):
        chunk = 2 * c + kk
        pltpu.sync_copy(zeros_h, accum.at[tsl])
        plsc.subcore_barrier()

        def fire(u):
            b = u % NBUF
            pltpu.async_copy(t_h.at[idx_v.at[u, 0]], bufs[b], gsem[b])

        def wait_g(u):
            b = u % NBUF
            pltpu.make_async_copy(
                t_h.at[idx_v.at[u, 0]], bufs[b], gsem[b]).wait()

        def scat(u):
            b = u % NBUF
            pltpu.async_copy(bufs[b], accum.at[idx_v.at[u, 1]], ssem[b],
                             add=True)

        def wait_s(u):
            b = u % NBUF
            pltpu.make_async_copy(
                bufs[b], accum.at[idx_v.at[u, 1]], ssem[b]).wait()

        def group(g, carry):
            pltpu.sync_copy(sd4.at[chunk, s, pl.ds(g * NUG, NUG)], idx_v)
            later = g > 0
            for x in range(DEPTH):
                @pl.when(later)
                def _(x=x):
                    wait_s(NUG - NBUF + x)

                fire(x)

            for u in range(NUG):
                nf = u + DEPTH
                if nf < NUG:
                    if u > 0:
                        wait_s(u - 1)
                    else:
                        @pl.when(later)
                        def _():
                            wait_s(NUG - 1)

                    fire(nf)
                wait_g(u)
                scat(u)
            return carry

        lax.fori_loop(0, NG_MAIN, group, 0)
        for u in range(NUG - NBUF, NUG):
            wait_s(u)
        plsc.subcore_barrier()
        pltpu.sync_copy(accum.at[tsl], o4.at[tsl, pl.ds(32 * chunk, 32)])


_BLK = 512
_GRID = NPAD // _BLK


def _dinv_block(degp):
    dsum = degp[0] + degp[1]
    return lax.rsqrt(dsum[:, 0:1] + 1.0)


_NAT = pl.BlockSpec((_BLK, D), lambda i: (i, 0))
_NAT_SHAPE = jax.ShapeDtypeStruct((NPAD, D), _f32)
_DEGP = pl.BlockSpec((NC, _BLK, 16), lambda i: (0, i, 0))


def _prescale_body(emb_ref, degp_ref, g):
    dinv = _dinv_block(degp_ref[...])
    g[...] = emb_ref[...] * dinv


def _tc_prescale(emb_pad, degp):
    return pl.pallas_call(
        _prescale_body,
        grid=(_GRID,),
        in_specs=[_NAT, _DEGP],
        out_specs=_NAT,
        out_shape=_NAT_SHAPE,
    )(emb_pad, degp)


def _mid_body(s, g, degp_ref, w1, b1, w2, f):
    dinv = _dinv_block(degp_ref[...])
    p1 = dinv * (s[...] + g[...])
    bf = jnp.bfloat16
    z = jnp.maximum(
        jnp.dot(p1.astype(bf), w1[...].astype(bf),
                preferred_element_type=_f32) + b1[...], 0.0)
    h2 = jnp.dot(z.astype(bf), w2[...].astype(bf),
                 preferred_element_type=_f32)
    f[...] = h2 * dinv


def _tc_mid(s, g, degp, W1, b1, W2):
    return pl.pallas_call(
        _mid_body,
        grid=(_GRID,),
        in_specs=[_NAT, _NAT, _DEGP,
                  pl.BlockSpec((D, 2 * D), lambda i: (0, 0)),
                  pl.BlockSpec((1, 2 * D), lambda i: (0, 0)),
                  pl.BlockSpec((2 * D, D), lambda i: (0, 0))],
        out_specs=_NAT,
        out_shape=_NAT_SHAPE,
    )(s, g, degp, W1, b1.reshape(1, 2 * D), W2)


def _final_body(s, f, degp_ref, b2, out):
    dinv = _dinv_block(degp_ref[...])
    out[...] = dinv * (s[...] + f[...]) + b2[...]


def _tc_final(s, f, degp, b2):
    return pl.pallas_call(
        _final_body,
        grid=(_GRID,),
        in_specs=[_NAT, _NAT, _DEGP,
                  pl.BlockSpec((1, D), lambda i: (0, 0))],
        out_specs=pl.BlockSpec((_BLK, D), lambda i: (i, 0)),
        out_shape=jax.ShapeDtypeStruct((N_NODES, D), _f32),
    )(s, f, degp, b2.reshape(1, D))


def kernel(edge_index, emb, W1, b1, W2, b2):
    src = edge_index[0].astype(jnp.int32)
    dst = edge_index[1].astype(jnp.int32)
    npad_extra = E_PAD - N_EDGES
    pad_ids = jnp.arange(npad_extra, dtype=jnp.int32)
    src_p = jnp.concatenate([src, pad_ids % N_NODES])
    dst_p = jnp.concatenate([dst, N_NODES + pad_ids % (NPAD - N_NODES)])
    src24 = (4 * src_p[None, :] + jnp.arange(4, dtype=jnp.int32)[:, None]
             ).reshape(4, NT, NU_MAIN, 1, UW)
    dst2 = jnp.broadcast_to(
        dst_p.reshape(1, NT, NU_MAIN, 1, UW), (4, NT, NU_MAIN, 1, UW))
    sd4 = jnp.concatenate([src24, dst2], axis=3)
    dst3 = dst_p.reshape(NT * NC, NB_DEG, 128)

    emb_pad = jnp.pad(emb, ((0, NPAD - N_NODES), (0, 0)))
    ones16 = jnp.ones((128, 16), _f32)
    z16 = jnp.zeros((ROWS_PER_TILE, 16), _f32)
    z32 = jnp.zeros((ROWS_PER_TILE, 32), _f32)

    degp = _deg_kernel(dst3, ones16, z16)
    g = _tc_prescale(emb_pad, degp)
    s = _scatter_kernel(g.reshape(4 * NPAD, 32), sd4, z32)
    f = _tc_mid(s, g, degp, W1, b1, W2)
    t = _scatter_kernel(f.reshape(4 * NPAD, 32), sd4, z32)
    return _tc_final(t, f, degp, b2)
